# Initial kernel scaffold; baseline (speedup 1.0000x reference)
import jax
import jax.numpy as jnp
from jax.experimental import pallas as pl


def kernel(hidden_states, gate_weight, e_score_correction_bias):
    raise NotImplementedError("write your pallas kernel here")


# speedup vs baseline: 2.2885x; 2.2885x over previous
import jax
import jax.numpy as jnp
from jax.experimental import pallas as pl

_E = 64
_K = 8
_NG = 8
_GS = 8
_TKG = 4
_SCALE = 2.5
_NEG = -1e30
_BIGI = 10_000


def _first_argmax(work, m, lane):
    return jnp.min(jnp.where(work == m, lane, _BIGI), axis=1, keepdims=True)


def _routing_kernel_body(h_ref, w_ref, b_ref, vals_ref, idx_ref):
    h = h_ref[...]
    w = w_ref[...]
    bias = b_ref[...]
    logits = jax.lax.dot_general(
        h, w, (((1,), (1,)), ((), ())),
        preferred_element_type=jnp.float32,
        precision=jax.lax.Precision.DEFAULT,
    )
    T = logits.shape[0]
    lane = jax.lax.broadcasted_iota(jnp.int32, (T, _E), 1)

    s = jax.nn.sigmoid(logits)
    swb = s + bias

    gexp = jnp.full((T, _E), _NEG, jnp.float32)
    for g in range(_NG):
        gmask = (lane >= g * _GS) & (lane < (g + 1) * _GS)
        vals = jnp.where(gmask, swb, _NEG)
        m1 = jnp.max(vals, axis=1, keepdims=True)
        i1 = _first_argmax(vals, m1, lane)
        m2 = jnp.max(jnp.where(lane == i1, _NEG, vals), axis=1, keepdims=True)
        gexp = jnp.where(gmask, m1 + m2, gexp)

    score_mask = jnp.zeros((T, _E), jnp.float32)
    gwork = gexp
    for _ in range(_TKG):
        m = jnp.max(gwork, axis=1, keepdims=True)
        i0 = _first_argmax(gwork, m, lane)
        g0 = (i0 // _GS) * _GS
        sel = (lane >= g0) & (lane < g0 + _GS)
        score_mask = jnp.where(sel, 1.0, score_mask)
        gwork = jnp.where(sel, _NEG, gwork)

    swb_m = swb * score_mask
    sel_mask = jnp.zeros((T, _E), jnp.bool_)
    work = swb_m
    for _ in range(_K):
        m = jnp.max(work, axis=1, keepdims=True)
        hit = lane == _first_argmax(work, m, lane)
        sel_mask = sel_mask | hit
        work = jnp.where(hit, _NEG, work)

    sm = jnp.where(sel_mask, s, 0.0)
    scale = _SCALE / (jnp.sum(sm, axis=1, keepdims=True) + 1e-20)
    t = jnp.where(sel_mask, s, _NEG)
    vals_cols = []
    idx_cols = []
    for _ in range(_K):
        m = jnp.max(t, axis=1, keepdims=True)
        i0 = _first_argmax(t, m, lane)
        vals_cols.append(m * scale)
        idx_cols.append(i0)
        t = jnp.where(lane == i0, _NEG, t)

    vals_ref[...] = jnp.concatenate(vals_cols, axis=1)
    idx_ref[...] = jnp.concatenate(idx_cols, axis=1).astype(jnp.int32)


def kernel(hidden_states, gate_weight, e_score_correction_bias):
    n_tok, hidden = hidden_states.shape
    t_tile = 256
    grid = (n_tok // t_tile,)
    bias2d = e_score_correction_bias.reshape(1, _E)
    vals, idxs = pl.pallas_call(
        _routing_kernel_body,
        grid=grid,
        in_specs=[
            pl.BlockSpec((t_tile, hidden), lambda i: (i, 0)),
            pl.BlockSpec((_E, hidden), lambda i: (0, 0)),
            pl.BlockSpec((1, _E), lambda i: (0, 0)),
        ],
        out_specs=[
            pl.BlockSpec((t_tile, _K), lambda i: (i, 0)),
            pl.BlockSpec((t_tile, _K), lambda i: (i, 0)),
        ],
        out_shape=[
            jax.ShapeDtypeStruct((n_tok, _K), jnp.float32),
            jax.ShapeDtypeStruct((n_tok, _K), jnp.int32),
        ],
    )(hidden_states, gate_weight, bias2d)
    return vals, idxs


# speedup vs baseline: 7.1439x; 3.1216x over previous
import jax
import jax.numpy as jnp
from jax.experimental import pallas as pl

_E = 64
_K = 8
_NG = 8
_GS = 8
_TKG = 4
_SCALE = 2.5
_NEG = -1e30
_BIG = 1e9


def _routing_kernel_body(h_ref, w_ref, b_ref, vals_ref, idx_ref):
    h = h_ref[...]
    w = w_ref[...]
    bias = b_ref[...]
    lt = jax.lax.dot_general(
        w, h, (((1,), (1,)), ((), ())),
        preferred_element_type=jnp.float32,
        precision=jax.lax.Precision.DEFAULT,
    )
    T = lt.shape[1]
    s2 = jax.nn.sigmoid(lt)
    swb2 = s2 + bias

    s = s2.reshape(_NG, _GS, T)
    v = swb2.reshape(_NG, _GS, T)
    sub_f = jax.lax.broadcasted_iota(jnp.int32, (_NG, _GS, T), 1).astype(jnp.float32)
    grp_f = jax.lax.broadcasted_iota(jnp.int32, (_NG, _GS, T), 0).astype(jnp.float32)
    eio_f = grp_f * float(_GS) + sub_f

    m1 = jnp.max(v, axis=1, keepdims=True)
    i1 = jnp.min(jnp.where(v == m1, sub_f, _BIG), axis=1, keepdims=True)
    m2 = jnp.max(jnp.where(sub_f == i1, _NEG, v), axis=1, keepdims=True)
    gwork = m1 + m2

    grp1_f = jax.lax.broadcasted_iota(jnp.int32, (_NG, 1, T), 0).astype(jnp.float32)
    gsel = jnp.zeros((_NG, 1, T), jnp.bool_)
    for _ in range(_TKG):
        m = jnp.max(gwork, axis=0, keepdims=True)
        gi = jnp.min(jnp.where(gwork == m, grp1_f, _BIG), axis=0, keepdims=True)
        hit = grp1_f == gi
        gsel = gsel | hit
        gwork = jnp.where(hit, _NEG, gwork)

    work = jnp.where(gsel, v, 0.0)

    selm = jnp.zeros((_NG, _GS, T), jnp.bool_)
    for _ in range(_K):
        m = jnp.max(work, axis=(0, 1), keepdims=True)
        ei = jnp.min(jnp.where(work == m, eio_f, _BIG), axis=(0, 1), keepdims=True)
        hit = eio_f == ei
        selm = selm | hit
        work = jnp.where(hit, _NEG, work)

    sm = jnp.where(selm, s, 0.0)
    scale = _SCALE / (jnp.sum(sm, axis=(0, 1), keepdims=True) + 1e-20)
    t = jnp.where(selm, s, _NEG)
    vals_rows = []
    idx_rows = []
    for _ in range(_K):
        m = jnp.max(t, axis=(0, 1), keepdims=True)
        ei = jnp.min(jnp.where(t == m, eio_f, _BIG), axis=(0, 1), keepdims=True)
        vals_rows.append(m * scale)
        idx_rows.append(ei)
        t = jnp.where(eio_f == ei, _NEG, t)

    vals_t = jnp.concatenate(vals_rows, axis=0).reshape(_K, T)
    idx_t = jnp.concatenate(idx_rows, axis=0).reshape(_K, T)
    vals_ref[...] = vals_t.T
    idx_ref[...] = idx_t.T.astype(jnp.int32)


def kernel(hidden_states, gate_weight, e_score_correction_bias):
    n_tok, hidden = hidden_states.shape
    t_tile = 256
    grid = (n_tok // t_tile,)
    bias_col = e_score_correction_bias.reshape(_E, 1)
    vals, idxs = pl.pallas_call(
        _routing_kernel_body,
        grid=grid,
        in_specs=[
            pl.BlockSpec((t_tile, hidden), lambda i: (i, 0)),
            pl.BlockSpec((_E, hidden), lambda i: (0, 0)),
            pl.BlockSpec((_E, 1), lambda i: (0, 0)),
        ],
        out_specs=[
            pl.BlockSpec((t_tile, _K), lambda i: (i, 0)),
            pl.BlockSpec((t_tile, _K), lambda i: (i, 0)),
        ],
        out_shape=[
            jax.ShapeDtypeStruct((n_tok, _K), jnp.float32),
            jax.ShapeDtypeStruct((n_tok, _K), jnp.int32),
        ],
    )(hidden_states, gate_weight, bias_col)
    return vals, idxs


# speedup vs baseline: 8.4023x; 1.1762x over previous
import jax
import jax.numpy as jnp
from jax.experimental import pallas as pl

_E = 64
_K = 8
_NG = 8
_GS = 8
_TKG = 4
_SCALE = 2.5
_NEG = -1e30
_BIG = 1e9


def _routing_kernel_body(h_ref, w_ref, b_ref, vals_ref, idx_ref):
    h = h_ref[...]
    w = w_ref[...]
    bias = b_ref[...]
    lt = jax.lax.dot_general(
        w, h, (((1,), (1,)), ((), ())),
        preferred_element_type=jnp.float32,
        precision=jax.lax.Precision.DEFAULT,
    )
    T = lt.shape[1]
    s2 = jax.nn.sigmoid(lt)
    swb2 = s2 + bias

    s = s2.reshape(_NG, _GS, T)
    v = swb2.reshape(_NG, _GS, T)
    sub_f = jax.lax.broadcasted_iota(jnp.int32, (_NG, _GS, T), 1).astype(jnp.float32)
    grp_f = jax.lax.broadcasted_iota(jnp.int32, (_NG, _GS, T), 0).astype(jnp.float32)
    eio_f = grp_f * float(_GS) + sub_f

    m1 = jnp.max(v, axis=1, keepdims=True)
    i1 = jnp.min(jnp.where(v == m1, sub_f, _BIG), axis=1, keepdims=True)
    m2 = jnp.max(jnp.where(sub_f == i1, _NEG, v), axis=1, keepdims=True)
    gwork = m1 + m2

    grp1_f = jax.lax.broadcasted_iota(jnp.int32, (_NG, 1, T), 0).astype(jnp.float32)
    gsel = jnp.zeros((_NG, 1, T), jnp.bool_)
    for _ in range(_TKG):
        m = jnp.max(gwork, axis=0, keepdims=True)
        gi = jnp.min(jnp.where(gwork == m, grp1_f, _BIG), axis=0, keepdims=True)
        hit = grp1_f == gi
        gsel = gsel | hit
        gwork = jnp.where(hit, _NEG, gwork)

    work = jnp.where(gsel, v, 0.0)

    selm = jnp.zeros((_NG, _GS, T), jnp.bool_)
    for _ in range(_K):
        m = jnp.max(work, axis=(0, 1), keepdims=True)
        ei = jnp.min(jnp.where(work == m, eio_f, _BIG), axis=(0, 1), keepdims=True)
        hit = eio_f == ei
        selm = selm | hit
        work = jnp.where(hit, _NEG, work)

    sm = jnp.where(selm, s, 0.0)
    scale = _SCALE / (jnp.sum(sm, axis=(0, 1), keepdims=True) + 1e-20)
    t = jnp.where(selm, s, _NEG)
    vals_rows = []
    idx_rows = []
    for _ in range(_K):
        m = jnp.max(t, axis=(0, 1), keepdims=True)
        ei = jnp.min(jnp.where(t == m, eio_f, _BIG), axis=(0, 1), keepdims=True)
        vals_rows.append(m * scale)
        idx_rows.append(ei)
        t = jnp.where(eio_f == ei, _NEG, t)

    vals_t = jnp.concatenate(vals_rows, axis=0).reshape(_K, T)
    idx_t = jnp.concatenate(idx_rows, axis=0).reshape(_K, T)
    vals_ref[...] = vals_t.T
    idx_ref[...] = idx_t.T.astype(jnp.int32)


def kernel(hidden_states, gate_weight, e_score_correction_bias):
    n_tok, hidden = hidden_states.shape
    t_tile = 512
    grid = (n_tok // t_tile,)
    bias_col = e_score_correction_bias.reshape(_E, 1)
    vals, idxs = pl.pallas_call(
        _routing_kernel_body,
        grid=grid,
        in_specs=[
            pl.BlockSpec((t_tile, hidden), lambda i: (i, 0)),
            pl.BlockSpec((_E, hidden), lambda i: (0, 0)),
            pl.BlockSpec((_E, 1), lambda i: (0, 0)),
        ],
        out_specs=[
            pl.BlockSpec((t_tile, _K), lambda i: (i, 0)),
            pl.BlockSpec((t_tile, _K), lambda i: (i, 0)),
        ],
        out_shape=[
            jax.ShapeDtypeStruct((n_tok, _K), jnp.float32),
            jax.ShapeDtypeStruct((n_tok, _K), jnp.int32),
        ],
    )(hidden_states, gate_weight, bias_col)
    return vals, idxs


# speedup vs baseline: 9.1264x; 1.0862x over previous
import jax
import jax.numpy as jnp
from jax.experimental import pallas as pl

_E = 64
_K = 8
_NG = 8
_GS = 8
_TKG = 4
_SCALE = 2.5
_NEG = -1e30
_BIG = 1e9


def _routing_kernel_body(h_ref, w_ref, b_ref, vals_ref, idx_ref):
    h = h_ref[...]
    w = w_ref[...]
    bias = b_ref[...]
    lt = jax.lax.dot_general(
        w, h, (((1,), (1,)), ((), ())),
        preferred_element_type=jnp.float32,
        precision=jax.lax.Precision.DEFAULT,
    )
    T = lt.shape[1]
    s2 = jax.nn.sigmoid(lt)
    swb2 = s2 + bias

    s = s2.reshape(_NG, _GS, T)
    v = swb2.reshape(_NG, _GS, T)
    sub_f = jax.lax.broadcasted_iota(jnp.int32, (_NG, _GS, T), 1).astype(jnp.float32)
    grp_f = jax.lax.broadcasted_iota(jnp.int32, (_NG, _GS, T), 0).astype(jnp.float32)
    eio_f = grp_f * float(_GS) + sub_f

    m1 = jnp.max(v, axis=1, keepdims=True)
    i1 = jnp.min(jnp.where(v == m1, sub_f, _BIG), axis=1, keepdims=True)
    m2 = jnp.max(jnp.where(sub_f == i1, _NEG, v), axis=1, keepdims=True)
    gwork = m1 + m2

    grp1_f = jax.lax.broadcasted_iota(jnp.int32, (_NG, 1, T), 0).astype(jnp.float32)
    gsel = jnp.zeros((_NG, 1, T), jnp.bool_)
    for _ in range(_TKG):
        m = jnp.max(gwork, axis=0, keepdims=True)
        gi = jnp.min(jnp.where(gwork == m, grp1_f, _BIG), axis=0, keepdims=True)
        hit = grp1_f == gi
        gsel = gsel | hit
        gwork = jnp.where(hit, _NEG, gwork)

    work = jnp.where(gsel, v, 0.0)

    selm = jnp.zeros((_NG, _GS, T), jnp.bool_)
    for _ in range(_K):
        m = jnp.max(work, axis=(0, 1), keepdims=True)
        ei = jnp.min(jnp.where(work == m, eio_f, _BIG), axis=(0, 1), keepdims=True)
        hit = eio_f == ei
        selm = selm | hit
        work = jnp.where(hit, _NEG, work)

    sm = jnp.where(selm, s, 0.0)
    scale = _SCALE / (jnp.sum(sm, axis=(0, 1), keepdims=True) + 1e-20)
    t = jnp.where(selm, s, _NEG)
    vals_rows = []
    idx_rows = []
    for _ in range(_K):
        m = jnp.max(t, axis=(0, 1), keepdims=True)
        ei = jnp.min(jnp.where(t == m, eio_f, _BIG), axis=(0, 1), keepdims=True)
        vals_rows.append(m * scale)
        idx_rows.append(ei)
        t = jnp.where(eio_f == ei, _NEG, t)

    vals_t = jnp.concatenate(vals_rows, axis=0).reshape(_K, T)
    idx_t = jnp.concatenate(idx_rows, axis=0).reshape(_K, T)
    vals_ref[...] = vals_t.T
    idx_ref[...] = idx_t.T.astype(jnp.int32)


def kernel(hidden_states, gate_weight, e_score_correction_bias):
    n_tok, hidden = hidden_states.shape
    t_tile = 1024
    grid = (n_tok // t_tile,)
    bias_col = e_score_correction_bias.reshape(_E, 1)
    vals, idxs = pl.pallas_call(
        _routing_kernel_body,
        grid=grid,
        in_specs=[
            pl.BlockSpec((t_tile, hidden), lambda i: (i, 0)),
            pl.BlockSpec((_E, hidden), lambda i: (0, 0)),
            pl.BlockSpec((_E, 1), lambda i: (0, 0)),
        ],
        out_specs=[
            pl.BlockSpec((t_tile, _K), lambda i: (i, 0)),
            pl.BlockSpec((t_tile, _K), lambda i: (i, 0)),
        ],
        out_shape=[
            jax.ShapeDtypeStruct((n_tok, _K), jnp.float32),
            jax.ShapeDtypeStruct((n_tok, _K), jnp.int32),
        ],
    )(hidden_states, gate_weight, bias_col)
    return vals, idxs
